# baseline (speedup 1.0000x reference)
import functools

import numpy as np
import jax
import jax.numpy as jnp
from jax import lax
from jax.experimental import pallas as pl
from jax.experimental.pallas import tpu as pltpu
from jax.experimental.pallas import tpu_sc as plsc

_B = 4096
_F = 26
_V = 100000
_D = 64
_NF = _F + 1
_NI = _NF * (_NF - 1) // 2
_H0, _H1 = 512, 256
_DENSE = 13

_NC = 2
_NS = 16
_NW = _NC * _NS
_CH = 128
_K = (_B * _F) // (_NW * _CH)

_BB = 256


def _sc_gather(table, idx3):
    mesh = plsc.VectorSubcoreMesh(core_axis_name="c", subcore_axis_name="s")
    nbuf = 4

    @functools.partial(
        pl.kernel,
        mesh=mesh,
        out_type=jax.ShapeDtypeStruct((_B * _F, 2 * _D), jnp.float32),
        scratch_types=(
            [pltpu.VMEM((_K, _CH), jnp.int32),
             pltpu.VMEM((nbuf, _CH, 2 * _D), jnp.float32)]
            + [pltpu.SemaphoreType.DMA] * (2 * nbuf)
        ),
        compiler_params=pltpu.CompilerParams(use_tc_tiling_on_sc=True),
    )
    def k(table_hbm, idx_hbm, out_hbm, idx_v, rows_v, *sems):
        gsems, ssems = sems[:nbuf], sems[nbuf:]
        wid = lax.axis_index("s") * _NC + lax.axis_index("c")
        pltpu.sync_copy(idx_hbm.at[wid], idx_v)
        base = wid * (_K * _CH)

        gd = [None] * _K
        sd = [None] * _K
        for j in range(_K + nbuf - 1):
            if j < _K:
                b = j % nbuf
                if j >= nbuf:
                    sd[j - nbuf].wait()
                gd[j] = pltpu.async_copy(
                    table_hbm.at[idx_v.at[j]], rows_v.at[b], gsems[b])
            i = j - (nbuf - 1)
            if i >= 0:
                gd[i].wait()
                sd[i] = pltpu.async_copy(
                    rows_v.at[i % nbuf],
                    out_hbm.at[pl.ds(base + i * _CH, _CH)],
                    ssems[i % nbuf])
        for i in range(_K - nbuf, _K):
            sd[i].wait()

    return k(table, idx3)


def _tc_body(xd, gv, par, bw0, bb0, bw1, bb1, bw2, bb2, w0z, w0d, tb0, tw1,
             tb1, tw2, tb2, out):
    f32 = jnp.float32
    h = jnp.maximum(jnp.dot(xd[...], bw0[...], preferred_element_type=f32) + bb0[...], 0.0)
    h = jnp.maximum(jnp.dot(h, bw1[...], preferred_element_type=f32) + bb1[...], 0.0)
    dense = jnp.dot(h, bw2[...], preferred_element_type=f32) + bb2[...]

    pair = gv[...]
    emb_rows = jnp.where(par[...][:, :, None] != 0,
                         pair[:, :, _D:], pair[:, :, :_D])
    c = jnp.concatenate([emb_rows, dense[:, None, :]], axis=1)
    zee = lax.dot_general(c, c, (((2,), (2,)), ((0,), (0,))),
                          preferred_element_type=f32)
    z = jnp.concatenate([zee[:, i, :] for i in range(_NF)], axis=1)

    t = (jnp.dot(z, w0z[...], preferred_element_type=f32)
         + jnp.dot(dense, w0d[...], preferred_element_type=f32) + tb0[...])
    t = jnp.maximum(t, 0.0)
    t = jnp.maximum(jnp.dot(t, tw1[...], preferred_element_type=f32) + tb1[...], 0.0)
    out[...] = jnp.dot(t, tw2[...], preferred_element_type=f32) + tb2[...]


def _tc_forward(x_dense, gv, par, bw0, bb0, bw1, bb1, bw2, bb2, w0z, w0d,
                tb0, tw1, tb1, tw2, tb2):
    nblk = _B // _BB

    def full(*shape):
        rank = len(shape)
        return pl.BlockSpec(shape, lambda i, _r=rank: (0,) * _r)

    grid_spec = pl.GridSpec(
        grid=(nblk,),
        in_specs=[
            pl.BlockSpec((_BB, _DENSE), lambda i: (i, 0)),
            pl.BlockSpec((_BB, _F, 2 * _D), lambda i: (i, 0, 0)),
            pl.BlockSpec((_BB, _F), lambda i: (i, 0)),
            full(_DENSE, _H0), full(1, _H0),
            full(_H0, _H1), full(1, _H1),
            full(_H1, _D), full(1, _D),
            full(_NF * _NF, _H0), full(_D, _H0), full(1, _H0),
            full(_H0, _H1), full(1, _H1),
            full(_H1, 1), full(1, 1),
        ],
        out_specs=pl.BlockSpec((_BB, 1), lambda i: (i, 0)),
    )
    return pl.pallas_call(
        _tc_body,
        grid_spec=grid_spec,
        out_shape=jax.ShapeDtypeStruct((_B, 1), jnp.float32),
    )(x_dense, gv, par, bw0, bb0, bw1, bb1, bw2, bb2, w0z, w0d, tb0, tw1,
      tb1, tw2, tb2)


def kernel(x_sparse, x_dense, emb, bw0, bb0, bw1, bb1, bw2, bb2, tw0, tb0,
           tw1, tb1, tw2, tb2):
    table = emb.reshape(_F * _V // 2, 2 * _D)
    flat_idx = (x_sparse.astype(jnp.int32)
                + (jnp.arange(_F, dtype=jnp.int32) * _V)[None, :])
    pair_idx = lax.shift_right_logical(flat_idx, 1)
    par = (flat_idx & 1).astype(jnp.float32)
    idx3 = pair_idx.reshape(_NW, _K, _CH)

    gv = _sc_gather(table, idx3).reshape(_B, _F, 2 * _D)

    iu, ju = np.triu_indices(_NF, k=1)
    w0z = jnp.zeros((_NF * _NF, _H0), jnp.float32).at[
        jnp.asarray(iu * _NF + ju)].set(tw0[:_NI])
    w0d = tw0[_NI:]

    return _tc_forward(
        x_dense, gv, par, bw0, bb0.reshape(1, -1), bw1, bb1.reshape(1, -1),
        bw2, bb2.reshape(1, -1), w0z, w0d, tb0.reshape(1, -1), tw1,
        tb1.reshape(1, -1), tw2, tb2.reshape(1, -1))


# speedup vs baseline: 1.0490x; 1.0490x over previous
import functools

import numpy as np
import jax
import jax.numpy as jnp
from jax import lax
from jax.experimental import pallas as pl
from jax.experimental.pallas import tpu as pltpu
from jax.experimental.pallas import tpu_sc as plsc

_B = 4096
_F = 26
_V = 100000
_D = 64
_NF = _F + 1
_NI = _NF * (_NF - 1) // 2
_H0, _H1 = 512, 256
_DENSE = 13

_NC = 2
_NS = 16
_NW = _NC * _NS
_CH = 128
_K = (_B * _F) // (_NW * _CH)

_BB = 256


def _sc_gather(table, idx3):
    mesh = plsc.VectorSubcoreMesh(core_axis_name="c", subcore_axis_name="s")
    nbuf = 4

    @functools.partial(
        pl.kernel,
        mesh=mesh,
        out_type=jax.ShapeDtypeStruct((_B * _F, _D), jnp.float32),
        scratch_types=(
            [pltpu.VMEM((_K, _CH), jnp.int32),
             pltpu.VMEM((nbuf, _CH, _D), jnp.float32)]
            + [pltpu.SemaphoreType.DMA] * (2 * nbuf)
        ),
        compiler_params=pltpu.CompilerParams(use_tc_tiling_on_sc=False),
    )
    def k(table_hbm, idx_hbm, out_hbm, idx_v, rows_v, *sems):
        gsems, ssems = sems[:nbuf], sems[nbuf:]
        wid = lax.axis_index("s") * _NC + lax.axis_index("c")
        pltpu.sync_copy(idx_hbm.at[wid], idx_v)
        base = wid * (_K * _CH)

        gd = [None] * _K
        sd = [None] * _K
        for j in range(_K + nbuf - 1):
            if j < _K:
                b = j % nbuf
                if j >= nbuf:
                    sd[j - nbuf].wait()
                gd[j] = pltpu.async_copy(
                    table_hbm.at[idx_v.at[j]], rows_v.at[b], gsems[b])
            i = j - (nbuf - 1)
            if i >= 0:
                gd[i].wait()
                sd[i] = pltpu.async_copy(
                    rows_v.at[i % nbuf],
                    out_hbm.at[pl.ds(base + i * _CH, _CH)],
                    ssems[i % nbuf])
        for i in range(_K - nbuf, _K):
            sd[i].wait()

    return k(table, idx3)


def _tc_body(xd, gv, bw0, bb0, bw1, bb1, bw2, bb2, w0z, w0d, tb0, tw1,
             tb1, tw2, tb2, out):
    f32 = jnp.float32
    h = jnp.maximum(jnp.dot(xd[...], bw0[...], preferred_element_type=f32) + bb0[...], 0.0)
    h = jnp.maximum(jnp.dot(h, bw1[...], preferred_element_type=f32) + bb1[...], 0.0)
    dense = jnp.dot(h, bw2[...], preferred_element_type=f32) + bb2[...]

    emb_rows = gv[...]
    c = jnp.concatenate([emb_rows, dense[:, None, :]], axis=1)
    zee = lax.dot_general(c, c, (((2,), (2,)), ((0,), (0,))),
                          preferred_element_type=f32)
    z = jnp.concatenate([zee[:, i, :] for i in range(_NF)], axis=1)

    t = (jnp.dot(z, w0z[...], preferred_element_type=f32)
         + jnp.dot(dense, w0d[...], preferred_element_type=f32) + tb0[...])
    t = jnp.maximum(t, 0.0)
    t = jnp.maximum(jnp.dot(t, tw1[...], preferred_element_type=f32) + tb1[...], 0.0)
    out[...] = jnp.dot(t, tw2[...], preferred_element_type=f32) + tb2[...]


def _tc_forward(x_dense, gv, bw0, bb0, bw1, bb1, bw2, bb2, w0z, w0d,
                tb0, tw1, tb1, tw2, tb2):
    nblk = _B // _BB

    def full(*shape):
        rank = len(shape)
        return pl.BlockSpec(shape, lambda i, _r=rank: (0,) * _r)

    grid_spec = pl.GridSpec(
        grid=(nblk,),
        in_specs=[
            pl.BlockSpec((_BB, _DENSE), lambda i: (i, 0)),
            pl.BlockSpec((_BB, _F, _D), lambda i: (i, 0, 0)),
            full(_DENSE, _H0), full(1, _H0),
            full(_H0, _H1), full(1, _H1),
            full(_H1, _D), full(1, _D),
            full(_NF * _NF, _H0), full(_D, _H0), full(1, _H0),
            full(_H0, _H1), full(1, _H1),
            full(_H1, 1), full(1, 1),
        ],
        out_specs=pl.BlockSpec((_BB, 1), lambda i: (i, 0)),
    )
    return pl.pallas_call(
        _tc_body,
        grid_spec=grid_spec,
        out_shape=jax.ShapeDtypeStruct((_B, 1), jnp.float32),
    )(x_dense, gv, bw0, bb0, bw1, bb1, bw2, bb2, w0z, w0d, tb0, tw1,
      tb1, tw2, tb2)


def kernel(x_sparse, x_dense, emb, bw0, bb0, bw1, bb1, bw2, bb2, tw0, tb0,
           tw1, tb1, tw2, tb2):
    table = emb.reshape(_F * _V, _D)
    flat_idx = (x_sparse.astype(jnp.int32)
                + (jnp.arange(_F, dtype=jnp.int32) * _V)[None, :])
    idx3 = flat_idx.reshape(_NW, _K, _CH)

    gv = _sc_gather(table, idx3).reshape(_B, _F, _D)

    iu, ju = np.triu_indices(_NF, k=1)
    w0z = jnp.zeros((_NF * _NF, _H0), jnp.float32).at[
        jnp.asarray(iu * _NF + ju)].set(tw0[:_NI])
    w0d = tw0[_NI:]

    return _tc_forward(
        x_dense, gv, bw0, bb0.reshape(1, -1), bw1, bb1.reshape(1, -1),
        bw2, bb2.reshape(1, -1), w0z, w0d, tb0.reshape(1, -1), tw1,
        tb1.reshape(1, -1), tw2, tb2.reshape(1, -1))
